# baseline (speedup 1.0000x reference)
import functools

import jax
import jax.numpy as jnp
from jax import lax
from jax.experimental import pallas as pl
from jax.experimental.pallas import tpu as pltpu
from jax.experimental.pallas import tpu_sc as plsc

T = 4096
H = 1024
NE = 64
FF = 512
FFS = 1024
NSLOT = 2 * NE
BT = 64
NB = 8192 // BT + NSLOT
NPAD = NB * BT

NC, NS = 2, 16
NW = NC * NS

@functools.cache
def _sc_mesh():
    return plsc.VectorSubcoreMesh(core_axis_name="c", subcore_axis_name="s",
                                  num_cores=NC, num_subcores=NS)


def _router_body(x_ref, gt_ref, gi_ref, m_ref, w_ref, s_ref):
    x = x_ref[...]
    lt = jnp.dot(x, gt_ref[...], preferred_element_type=jnp.float32)
    li = jnp.dot(x, gi_ref[...], preferred_element_type=jnp.float32)
    m = m_ref[...]
    logits = jnp.where(m > 0, li, lt)
    z = logits - jnp.max(logits, axis=-1, keepdims=True)
    ez = jnp.exp(z)
    p = ez / jnp.sum(ez, axis=-1, keepdims=True)
    iota = lax.broadcasted_iota(jnp.int32, p.shape, 1)
    big = jnp.int32(1 << 30)
    m1 = jnp.max(p, axis=-1, keepdims=True)
    i1 = jnp.min(jnp.where(p == m1, iota, big), axis=-1, keepdims=True)
    pm = jnp.where(iota == i1, -jnp.inf, p)
    m2 = jnp.max(pm, axis=-1, keepdims=True)
    i2 = jnp.min(jnp.where(pm == m2, iota, big), axis=-1, keepdims=True)
    tot = m1 + m2
    off = m * NE
    oi = lax.broadcasted_iota(jnp.int32, w_ref.shape, 1)
    w_ref[...] = jnp.where(oi == 0, m1 / tot, jnp.where(oi == 1, m2 / tot, 0.0))
    s_ref[...] = jnp.where(oi == 0, i1 + off, jnp.where(oi == 1, i2 + off, 0))


def _router(x, gate_t, gate_i, mask_i):
    rb = 1024
    grid = T // rb
    return pl.pallas_call(
        _router_body,
        grid=(grid,),
        in_specs=[
            pl.BlockSpec((rb, H), lambda i: (i, 0)),
            pl.BlockSpec((H, NE), lambda i: (0, 0)),
            pl.BlockSpec((H, NE), lambda i: (0, 0)),
            pl.BlockSpec((rb, 1), lambda i: (i, 0)),
        ],
        out_specs=[
            pl.BlockSpec((rb, 128), lambda i: (i, 0)),
            pl.BlockSpec((rb, 128), lambda i: (i, 0)),
        ],
        out_shape=[
            jax.ShapeDtypeStruct((T, 128), jnp.float32),
            jax.ShapeDtypeStruct((T, 128), jnp.int32),
        ],
    )(x, gate_t, gate_i, mask_i)


_G_S = 8
_G_RC = 8
_G_ROWS_W = NPAD // NW
_G_NR = _G_ROWS_W // (_G_S * _G_RC)


def _sc_gather_body(x_hbm, src_hbm, out_hbm, ibufs, rbufs, gsem, wsem, isem):
    wid = lax.axis_index("s") * NC + lax.axis_index("c")
    base0 = wid * _G_ROWS_W

    def fire_idx(r, pp):
        hs = []
        for j in range(_G_S):
            off = base0 + (r * _G_S + j) * _G_RC
            hs.append(pltpu.async_copy(
                src_hbm.at[pl.ds(off, _G_RC)], ibufs[pp][j], isem))
        return hs

    hi = fire_idx(0, 0)
    for r in range(_G_NR):
        pp = r % 2
        for h in hi:
            h.wait()
        hg = [pltpu.async_copy(x_hbm.at[ibufs[pp][j]], rbufs[j], gsem)
              for j in range(_G_S)]
        if r + 1 < _G_NR:
            hi = fire_idx(r + 1, 1 - pp)
        for h in hg:
            h.wait()
        hw = [pltpu.async_copy(
                  rbufs[j],
                  out_hbm.at[pl.ds(base0 + (r * _G_S + j) * _G_RC, _G_RC)],
                  wsem)
              for j in range(_G_S)]
        for h in hw:
            h.wait()


@functools.cache
def _sc_gather():
    return pl.kernel(
        _sc_gather_body,
        mesh=_sc_mesh(),
        out_type=jax.ShapeDtypeStruct((NPAD, H), jnp.float32),
        scratch_types=[
            [[pltpu.VMEM((_G_RC,), jnp.int32) for _ in range(_G_S)]
             for _ in range(2)],
            [pltpu.VMEM((_G_RC, H), jnp.float32) for _ in range(_G_S)],
            pltpu.SemaphoreType.DMA, pltpu.SemaphoreType.DMA,
            pltpu.SemaphoreType.DMA,
        ],
    )


def _moe_body(sid_ref, nf_ref, bp_ref, nbu_ref, x_ref, wr_ref,
              wgt_r, wut_r, wdt_r, wgi_r, wui_r, wdi_r, y_ref,
              wg0, wg1, wu0, wu1, wd0, wd1, sg0, sg1, su0, su1, sd0, sd1):
    i = pl.program_id(0)

    def start(blk):
        s = sid_ref[blk]
        b = bp_ref[blk]

        def issue(wg, wu, wd, sg, su, sd):
            @pl.when(s < NE)
            def _():
                pltpu.make_async_copy(wgt_r.at[s], wg, sg).start()
                pltpu.make_async_copy(wut_r.at[s], wu, su).start()
                pltpu.make_async_copy(wdt_r.at[s], wd, sd).start()

            @pl.when(s >= NE)
            def _():
                pltpu.make_async_copy(wgi_r.at[s - NE], wg, sg).start()
                pltpu.make_async_copy(wui_r.at[s - NE], wu, su).start()
                pltpu.make_async_copy(wdi_r.at[s - NE], wd, sd).start()

        @pl.when(b == 0)
        def _():
            issue(wg0, wu0, wd0, sg0, su0, sd0)

        @pl.when(b == 1)
        def _():
            issue(wg1, wu1, wd1, sg1, su1, sd1)

    @pl.when(i == 0)
    def _():
        start(0)

    nxt = jnp.minimum(i + 1, NB - 1)

    @pl.when((i + 1 < NB) & (nf_ref[nxt] == 1))
    def _():
        start(nxt)

    b = bp_ref[i]

    def wait(wg, wu, wd, sg, su, sd):
        pltpu.make_async_copy(wgt_r.at[0], wg, sg).wait()
        pltpu.make_async_copy(wut_r.at[0], wu, su).wait()
        pltpu.make_async_copy(wdt_r.at[0], wd, sd).wait()

    @pl.when((nf_ref[i] == 1) & (b == 0))
    def _():
        wait(wg0, wu0, wd0, sg0, su0, sd0)

    @pl.when((nf_ref[i] == 1) & (b == 1))
    def _():
        wait(wg1, wu1, wd1, sg1, su1, sd1)

    def compute(wg, wu, wd):
        x = x_ref[...]
        g = jnp.dot(x, wg[...], preferred_element_type=jnp.float32)
        u = jnp.dot(x, wu[...], preferred_element_type=jnp.float32)
        h = g * (1.0 / (1.0 + jnp.exp(-g))) * u
        y = jnp.dot(h, wd[...], preferred_element_type=jnp.float32)
        y_ref[...] = y * wr_ref[...]

    @pl.when((i < nbu_ref[0]) & (b == 0))
    def _():
        compute(wg0, wu0, wd0)

    @pl.when((i < nbu_ref[0]) & (b == 1))
    def _():
        compute(wg1, wu1, wd1)


def _grouped_mlp(blk_sid, nf, bufp, nbu, x_pad, wrow,
                 Wg_t, Wu_t, Wd_t, Wg_i, Wu_i, Wd_i):
    smem = pl.BlockSpec(memory_space=pltpu.MemorySpace.SMEM)
    anym = pl.BlockSpec(memory_space=pltpu.MemorySpace.HBM)
    return pl.pallas_call(
        _moe_body,
        grid=(NB,),
        in_specs=[
            smem, smem, smem, smem,
            pl.BlockSpec((BT, H), lambda i: (i, 0)),
            pl.BlockSpec((BT, 1), lambda i: (i, 0)),
            anym, anym, anym, anym, anym, anym,
        ],
        out_specs=pl.BlockSpec((BT, H), lambda i: (i, 0)),
        out_shape=jax.ShapeDtypeStruct((NPAD, H), jnp.float32),
        scratch_shapes=[
            pltpu.VMEM((H, FF), jnp.float32), pltpu.VMEM((H, FF), jnp.float32),
            pltpu.VMEM((H, FF), jnp.float32), pltpu.VMEM((H, FF), jnp.float32),
            pltpu.VMEM((FF, H), jnp.float32), pltpu.VMEM((FF, H), jnp.float32),
            pltpu.SemaphoreType.DMA, pltpu.SemaphoreType.DMA,
            pltpu.SemaphoreType.DMA, pltpu.SemaphoreType.DMA,
            pltpu.SemaphoreType.DMA, pltpu.SemaphoreType.DMA,
        ],
    )(blk_sid, nf, bufp, nbu, x_pad, wrow, Wg_t, Wu_t, Wd_t, Wg_i, Wu_i, Wd_i)


def _shared_body(x_ref, wg_ref, wu_ref, wd_ref, o_ref):
    x = x_ref[...].astype(jnp.bfloat16)
    g = jnp.dot(x, wg_ref[...], preferred_element_type=jnp.float32)
    u = jnp.dot(x, wu_ref[...], preferred_element_type=jnp.float32)
    h = (g * (1.0 / (1.0 + jnp.exp(-g))) * u).astype(jnp.bfloat16)
    o_ref[...] = jnp.dot(h, wd_ref[...], preferred_element_type=jnp.float32)


def _shared_mlp(x, Wg_s, Wu_s, Wd_s):
    rb = 512
    f2 = 2 * FFS
    return pl.pallas_call(
        _shared_body,
        grid=(T // rb,),
        in_specs=[
            pl.BlockSpec((rb, H), lambda i: (i, 0)),
            pl.BlockSpec((H, f2), lambda i: (0, 0)),
            pl.BlockSpec((H, f2), lambda i: (0, 0)),
            pl.BlockSpec((f2, H), lambda i: (0, 0)),
        ],
        out_specs=pl.BlockSpec((rb, H), lambda i: (i, 0)),
        out_shape=jax.ShapeDtypeStruct((T, H), jnp.float32),
    )(x, Wg_s.astype(jnp.bfloat16), Wu_s.astype(jnp.bfloat16),
      Wd_s.astype(jnp.bfloat16))


def _sc_combine_body(y_hbm, sh_hbm, p0_hbm, p1_hbm, out_hbm,
                     i0_v, i1_v, y0_v, y1_v, sh_v, sem0, sem1):
    wid = lax.axis_index("s") * NC + lax.axis_index("c")
    tok_w = T // NW
    ch = 16
    base0 = wid * tok_w
    for c in range(tok_w // ch):
        base = base0 + c * ch
        pltpu.sync_copy(p0_hbm.at[pl.ds(base, ch)], i0_v)
        pltpu.sync_copy(p1_hbm.at[pl.ds(base, ch)], i1_v)
        cp0 = pltpu.async_copy(y_hbm.at[i0_v], y0_v, sem0)
        cp1 = pltpu.async_copy(y_hbm.at[i1_v], y1_v, sem1)
        pltpu.sync_copy(sh_hbm.at[pl.ds(base, ch)], sh_v)
        cp0.wait()
        cp1.wait()
        for r in range(ch):
            def col(k, carry):
                sl = pl.ds(k * 16, 16)
                y0_v[r, sl] = y0_v[r, sl] + y1_v[r, sl] + sh_v[r, sl]
                return carry
            lax.fori_loop(0, H // 16, col, 0)
        pltpu.sync_copy(y0_v, out_hbm.at[pl.ds(base, ch)])


@functools.cache
def _sc_combine():
    return pl.kernel(
        _sc_combine_body,
        mesh=_sc_mesh(),
        out_type=jax.ShapeDtypeStruct((T, H), jnp.float32),
        scratch_types=[
            pltpu.VMEM((16,), jnp.int32), pltpu.VMEM((16,), jnp.int32),
            pltpu.VMEM((16, H), jnp.float32), pltpu.VMEM((16, H), jnp.float32),
            pltpu.VMEM((16, H), jnp.float32),
            pltpu.SemaphoreType.DMA, pltpu.SemaphoreType.DMA,
        ],
    )


def kernel(hidden_states, visual_token_mask, gate_t, Wg_t, Wu_t, Wd_t,
           gate_i, Wg_i, Wu_i, Wd_i, Wg_s, Wu_s, Wd_s):
    orig = hidden_states.shape
    x = hidden_states.reshape(T, H)
    mask_i = visual_token_mask.reshape(T, 1).astype(jnp.int32)

    w_out, s_out = _router(x, gate_t, gate_i, mask_i)
    wgt = w_out[:, :2].reshape(-1)
    sid = s_out[:, :2].reshape(-1)

    ch = 128
    nchk = 8192 // ch
    ar_s = jnp.arange(NSLOT, dtype=jnp.int32)
    ohc = (sid.reshape(nchk, ch, 1) == ar_s).astype(jnp.bfloat16)
    tril = jnp.tril(jnp.ones((ch, ch), jnp.float32), -1).astype(jnp.bfloat16)
    within = jnp.einsum("ij,cjs->cis", tril, ohc,
                        preferred_element_type=jnp.float32)
    chunk_tot = ohc.astype(jnp.float32).sum(axis=1)
    offs_c = jnp.cumsum(chunk_tot, axis=0) - chunk_tot
    rank = jnp.sum((within + offs_c[:, None, :]) * ohc.astype(jnp.float32),
                   axis=2).reshape(8192).astype(jnp.int32)
    counts = (offs_c[-1] + chunk_tot[-1]).astype(jnp.int32)

    nblk = (counts + BT - 1) // BT
    blk_end = jnp.cumsum(nblk)
    pad_off = (blk_end - nblk) * BT
    cnt_ex = jnp.cumsum(counts) - counts
    dest = pad_off[sid] + rank
    pos = dest.reshape(T, 2)

    total_blocks = blk_end[NSLOT - 1]
    bi = jnp.arange(NB, dtype=jnp.int32)
    ss = jnp.searchsorted(blk_end, bi, side="right").astype(jnp.int32)
    valid_b = bi < total_blocks
    last_slot = jnp.max(jnp.where(valid_b, ss, -1))
    blk_sid = jnp.where(valid_b, ss, last_slot).astype(jnp.int32)

    tok = (jnp.arange(8192, dtype=jnp.int32) >> 1)
    cpos = cnt_ex[sid] + rank
    inv = jnp.zeros((8192,), jnp.int32).at[cpos].set(
        jnp.arange(8192, dtype=jnp.int32), unique_indices=True)
    rows = jnp.arange(NPAD, dtype=jnp.int32)
    s_r = jnp.broadcast_to(blk_sid[:, None], (NB, BT)).reshape(-1)
    k_r = rows - pad_off[s_r]
    valid_r = k_r < counts[s_r]
    cidx = jnp.clip(cnt_ex[s_r] + k_r, 0, 8191)
    aidx = inv[cidx]
    src = jnp.where(valid_r, tok[aidx], (rows * 8) % T)
    wrow = jnp.where(valid_r, wgt[aidx], 0.0)[:, None]
    nf = jnp.concatenate(
        [jnp.ones((1,), jnp.int32),
         (blk_sid[1:] != blk_sid[:-1]).astype(jnp.int32)])
    bufp = (jnp.cumsum(nf) - 1) % 2
    nbu = jnp.reshape(total_blocks, (1,)).astype(jnp.int32)

    x_pad = _sc_gather()(x, src)
    y_pad = _grouped_mlp(blk_sid, nf, bufp, nbu, x_pad, wrow,
                         Wg_t, Wu_t, Wd_t, Wg_i, Wu_i, Wd_i)
    shared = _shared_mlp(x, Wg_s, Wu_s, Wd_s)
    out = _sc_combine()(y_pad, shared, pos[:, 0], pos[:, 1])
    return out.reshape(orig)


# speedup vs baseline: 2.1793x; 2.1793x over previous
import functools

import jax
import jax.numpy as jnp
from jax import lax
from jax.experimental import pallas as pl
from jax.experimental.pallas import tpu as pltpu
from jax.experimental.pallas import tpu_sc as plsc

T = 4096
H = 1024
NE = 64
FF = 512
FFS = 1024
NSLOT = 2 * NE
BT = 64
NB = 8192 // BT + NSLOT
NPAD = NB * BT

NC, NS = 2, 16
NW = NC * NS

@functools.cache
def _sc_mesh():
    return plsc.VectorSubcoreMesh(core_axis_name="c", subcore_axis_name="s",
                                  num_cores=NC, num_subcores=NS)


def _router_body(x_ref, gt_ref, gi_ref, m_ref, w_ref, s_ref):
    x = x_ref[...]
    lt = jnp.dot(x, gt_ref[...], preferred_element_type=jnp.float32)
    li = jnp.dot(x, gi_ref[...], preferred_element_type=jnp.float32)
    m = m_ref[...]
    logits = jnp.where(m > 0, li, lt)
    z = logits - jnp.max(logits, axis=-1, keepdims=True)
    ez = jnp.exp(z)
    p = ez / jnp.sum(ez, axis=-1, keepdims=True)
    iota = lax.broadcasted_iota(jnp.int32, p.shape, 1)
    big = jnp.int32(1 << 30)
    m1 = jnp.max(p, axis=-1, keepdims=True)
    i1 = jnp.min(jnp.where(p == m1, iota, big), axis=-1, keepdims=True)
    pm = jnp.where(iota == i1, -jnp.inf, p)
    m2 = jnp.max(pm, axis=-1, keepdims=True)
    i2 = jnp.min(jnp.where(pm == m2, iota, big), axis=-1, keepdims=True)
    tot = m1 + m2
    off = m * NE
    oi = lax.broadcasted_iota(jnp.int32, w_ref.shape, 1)
    w_ref[...] = jnp.where(oi == 0, m1 / tot, jnp.where(oi == 1, m2 / tot, 0.0))
    s_ref[...] = jnp.where(oi == 0, i1 + off, jnp.where(oi == 1, i2 + off, 0))


def _router(x, gate_t, gate_i, mask_i):
    rb = 1024
    grid = T // rb
    return pl.pallas_call(
        _router_body,
        grid=(grid,),
        in_specs=[
            pl.BlockSpec((rb, H), lambda i: (i, 0)),
            pl.BlockSpec((H, NE), lambda i: (0, 0)),
            pl.BlockSpec((H, NE), lambda i: (0, 0)),
            pl.BlockSpec((rb, 1), lambda i: (i, 0)),
        ],
        out_specs=[
            pl.BlockSpec((rb, 128), lambda i: (i, 0)),
            pl.BlockSpec((rb, 128), lambda i: (i, 0)),
        ],
        out_shape=[
            jax.ShapeDtypeStruct((T, 128), jnp.float32),
            jax.ShapeDtypeStruct((T, 128), jnp.int32),
        ],
    )(x, gate_t, gate_i, mask_i)


_G_S = 8
_G_RC = 8
_G_ROWS_W = NPAD // NW
_G_NR = _G_ROWS_W // (_G_S * _G_RC)


def _sc_gather_body(x_hbm, src_hbm, out_hbm, ibufs, rbufs, gsem, wsem, isem):
    wid = lax.axis_index("s") * NC + lax.axis_index("c")
    base0 = wid * _G_ROWS_W

    def fire_idx(r, pp):
        hs = []
        for j in range(_G_S):
            off = base0 + (r * _G_S + j) * _G_RC
            hs.append(pltpu.async_copy(
                src_hbm.at[pl.ds(off, _G_RC)], ibufs[pp][j], isem))
        return hs

    hi = fire_idx(0, 0)
    for r in range(_G_NR):
        pp = r % 2
        for h in hi:
            h.wait()
        hg = [pltpu.async_copy(x_hbm.at[ibufs[pp][j]], rbufs[j], gsem)
              for j in range(_G_S)]
        if r + 1 < _G_NR:
            hi = fire_idx(r + 1, 1 - pp)
        for h in hg:
            h.wait()
        hw = [pltpu.async_copy(
                  rbufs[j],
                  out_hbm.at[pl.ds(base0 + (r * _G_S + j) * _G_RC, _G_RC)],
                  wsem)
              for j in range(_G_S)]
        for h in hw:
            h.wait()


@functools.cache
def _sc_gather():
    return pl.kernel(
        _sc_gather_body,
        mesh=_sc_mesh(),
        out_type=jax.ShapeDtypeStruct((NPAD, H), jnp.float32),
        scratch_types=[
            [[pltpu.VMEM((_G_RC,), jnp.int32) for _ in range(_G_S)]
             for _ in range(2)],
            [pltpu.VMEM((_G_RC, H), jnp.float32) for _ in range(_G_S)],
            pltpu.SemaphoreType.DMA, pltpu.SemaphoreType.DMA,
            pltpu.SemaphoreType.DMA,
        ],
    )


_D = 4


def _moe_body(nf_ref, buf_ref, nbu_ref, nfetch_ref, iv_ref, islot_ref,
              ibuf_ref, prslot_ref, x_ref, wr_ref,
              wgt_r, wut_r, wdt_r, wgi_r, wui_r, wdi_r, y_ref, *scr):
    wgb = scr[0:_D]
    wub = scr[_D:2 * _D]
    wdb = scr[2 * _D:3 * _D]
    sg = scr[3 * _D:4 * _D]
    su = scr[4 * _D:5 * _D]
    sd = scr[5 * _D:6 * _D]
    i = pl.program_id(0)

    def issue(slot, d):
        @pl.when(slot < NE)
        def _():
            pltpu.make_async_copy(wgt_r.at[slot], wgb[d], sg[d]).start()
            pltpu.make_async_copy(wut_r.at[slot], wub[d], su[d]).start()
            pltpu.make_async_copy(wdt_r.at[slot], wdb[d], sd[d]).start()

        @pl.when(slot >= NE)
        def _():
            pltpu.make_async_copy(wgi_r.at[slot - NE], wgb[d], sg[d]).start()
            pltpu.make_async_copy(wui_r.at[slot - NE], wub[d], su[d]).start()
            pltpu.make_async_copy(wdi_r.at[slot - NE], wdb[d], sd[d]).start()

    @pl.when(i == 0)
    def _():
        for f in range(_D - 1):
            @pl.when(f < nfetch_ref[0])
            def _(f=f):
                issue(prslot_ref[f], f)

    @pl.when(iv_ref[i] == 1)
    def _():
        for d in range(_D):
            @pl.when(ibuf_ref[i] == d)
            def _(d=d):
                issue(islot_ref[i], d)

    @pl.when(nf_ref[i] == 1)
    def _():
        for d in range(_D):
            @pl.when(buf_ref[i] == d)
            def _(d=d):
                pltpu.make_async_copy(wgt_r.at[0], wgb[d], sg[d]).wait()
                pltpu.make_async_copy(wut_r.at[0], wub[d], su[d]).wait()
                pltpu.make_async_copy(wdt_r.at[0], wdb[d], sd[d]).wait()

    def compute(wg, wu, wd):
        x = x_ref[...]
        g = jnp.dot(x, wg[...], preferred_element_type=jnp.float32)
        u = jnp.dot(x, wu[...], preferred_element_type=jnp.float32)
        h = g * (1.0 / (1.0 + jnp.exp(-g))) * u
        y = jnp.dot(h, wd[...], preferred_element_type=jnp.float32)
        y_ref[...] = y * wr_ref[...]

    @pl.when(i < nbu_ref[0])
    def _():
        for d in range(_D):
            @pl.when(buf_ref[i] == d)
            def _(d=d):
                compute(wgb[d], wub[d], wdb[d])


def _grouped_mlp(nf, buf, nbu, nfetch, iv, islot, ibuf, prslot, x_pad, wrow,
                 Wg_t, Wu_t, Wd_t, Wg_i, Wu_i, Wd_i):
    smem = pl.BlockSpec(memory_space=pltpu.MemorySpace.SMEM)
    anym = pl.BlockSpec(memory_space=pltpu.MemorySpace.HBM)
    return pl.pallas_call(
        _moe_body,
        grid=(NB,),
        in_specs=[
            smem, smem, smem, smem, smem, smem, smem, smem,
            pl.BlockSpec((BT, H), lambda i: (i, 0)),
            pl.BlockSpec((BT, 1), lambda i: (i, 0)),
            anym, anym, anym, anym, anym, anym,
        ],
        out_specs=pl.BlockSpec((BT, H), lambda i: (i, 0)),
        out_shape=jax.ShapeDtypeStruct((NPAD, H), jnp.float32),
        scratch_shapes=(
            [pltpu.VMEM((H, FF), jnp.float32) for _ in range(_D)]
            + [pltpu.VMEM((H, FF), jnp.float32) for _ in range(_D)]
            + [pltpu.VMEM((FF, H), jnp.float32) for _ in range(_D)]
            + [pltpu.SemaphoreType.DMA for _ in range(3 * _D)]
        ),
    )(nf, buf, nbu, nfetch, iv, islot, ibuf, prslot, x_pad, wrow,
      Wg_t, Wu_t, Wd_t, Wg_i, Wu_i, Wd_i)


def _shared_body(x_ref, wg_ref, wu_ref, wd_ref, o_ref):
    x = x_ref[...].astype(jnp.bfloat16)
    g = jnp.dot(x, wg_ref[...], preferred_element_type=jnp.float32)
    u = jnp.dot(x, wu_ref[...], preferred_element_type=jnp.float32)
    h = (g * (1.0 / (1.0 + jnp.exp(-g))) * u).astype(jnp.bfloat16)
    o_ref[...] = jnp.dot(h, wd_ref[...], preferred_element_type=jnp.float32)


def _shared_mlp(x, Wg_s, Wu_s, Wd_s):
    rb = 512
    f2 = 2 * FFS
    return pl.pallas_call(
        _shared_body,
        grid=(T // rb,),
        in_specs=[
            pl.BlockSpec((rb, H), lambda i: (i, 0)),
            pl.BlockSpec((H, f2), lambda i: (0, 0)),
            pl.BlockSpec((H, f2), lambda i: (0, 0)),
            pl.BlockSpec((f2, H), lambda i: (0, 0)),
        ],
        out_specs=pl.BlockSpec((rb, H), lambda i: (i, 0)),
        out_shape=jax.ShapeDtypeStruct((T, H), jnp.float32),
    )(x, Wg_s.astype(jnp.bfloat16), Wu_s.astype(jnp.bfloat16),
      Wd_s.astype(jnp.bfloat16))


def _sc_combine_body(y_hbm, sh_hbm, p0_hbm, p1_hbm, out_hbm,
                     i0_v, i1_v, y0_v, y1_v, sh_v, sem0, sem1):
    wid = lax.axis_index("s") * NC + lax.axis_index("c")
    tok_w = T // NW
    ch = 16
    base0 = wid * tok_w
    for c in range(tok_w // ch):
        base = base0 + c * ch
        pltpu.sync_copy(p0_hbm.at[pl.ds(base, ch)], i0_v)
        pltpu.sync_copy(p1_hbm.at[pl.ds(base, ch)], i1_v)
        cp0 = pltpu.async_copy(y_hbm.at[i0_v], y0_v, sem0)
        cp1 = pltpu.async_copy(y_hbm.at[i1_v], y1_v, sem1)
        pltpu.sync_copy(sh_hbm.at[pl.ds(base, ch)], sh_v)
        cp0.wait()
        cp1.wait()
        for r in range(ch):
            def col(k, carry):
                sl = pl.ds(k * 16, 16)
                y0_v[r, sl] = y0_v[r, sl] + y1_v[r, sl] + sh_v[r, sl]
                return carry
            lax.fori_loop(0, H // 16, col, 0)
        pltpu.sync_copy(y0_v, out_hbm.at[pl.ds(base, ch)])


@functools.cache
def _sc_combine():
    return pl.kernel(
        _sc_combine_body,
        mesh=_sc_mesh(),
        out_type=jax.ShapeDtypeStruct((T, H), jnp.float32),
        scratch_types=[
            pltpu.VMEM((16,), jnp.int32), pltpu.VMEM((16,), jnp.int32),
            pltpu.VMEM((16, H), jnp.float32), pltpu.VMEM((16, H), jnp.float32),
            pltpu.VMEM((16, H), jnp.float32),
            pltpu.SemaphoreType.DMA, pltpu.SemaphoreType.DMA,
        ],
    )


def kernel(hidden_states, visual_token_mask, gate_t, Wg_t, Wu_t, Wd_t,
           gate_i, Wg_i, Wu_i, Wd_i, Wg_s, Wu_s, Wd_s):
    orig = hidden_states.shape
    x = hidden_states.reshape(T, H)
    mask_i = visual_token_mask.reshape(T, 1).astype(jnp.int32)

    w_out, s_out = _router(x, gate_t, gate_i, mask_i)
    wgt = w_out[:, :2].reshape(-1)
    sid = s_out[:, :2].reshape(-1)

    oh = (sid[:, None] == jnp.arange(NSLOT, dtype=jnp.int32)[None, :])
    ohi = oh.astype(jnp.int32)
    counts = ohi.sum(axis=0)
    rank = jnp.sum((jnp.cumsum(ohi, axis=0) - ohi) * ohi, axis=1)
    nblk = (counts + BT - 1) // BT
    blk_end = jnp.cumsum(nblk)
    pad_off = (blk_end - nblk) * BT
    dest = pad_off[sid] + rank

    tok = (jnp.arange(8192, dtype=jnp.int32) >> 1)
    pad_spread = (jnp.arange(NPAD, dtype=jnp.int32) * 8) % T
    src = pad_spread.at[dest].set(tok)
    wrow = jnp.zeros((NPAD, 1), jnp.float32).at[dest, 0].set(wgt)
    pos = dest.reshape(T, 2)

    total_blocks = blk_end[NSLOT - 1]
    bi = jnp.arange(NB, dtype=jnp.int32)
    ss = jnp.searchsorted(blk_end, bi, side="right").astype(jnp.int32)
    valid_b = bi < total_blocks
    last_slot = jnp.max(jnp.where(valid_b, ss, -1))
    blk_sid = jnp.where(valid_b, ss, last_slot).astype(jnp.int32)
    nf = jnp.concatenate(
        [jnp.ones((1,), jnp.int32),
         (blk_sid[1:] != blk_sid[:-1]).astype(jnp.int32)])
    nbu = jnp.reshape(total_blocks, (1,)).astype(jnp.int32)

    fc = jnp.cumsum(nf)
    fidx = fc - 1
    nfetch = jnp.reshape(fc[NB - 1], (1,)).astype(jnp.int32)
    fslot = jnp.zeros((NB,), jnp.int32).at[
        jnp.where(nf == 1, fidx, NB)].set(blk_sid, mode="drop")
    issue_f = fidx + (_D - 1)
    iv = ((nf == 1) & (issue_f < fc[NB - 1])).astype(jnp.int32)
    islot = fslot[jnp.clip(issue_f, 0, NB - 1)]
    ibuf = issue_f % _D
    buf = fidx % _D
    prslot = fslot[: _D - 1]

    x_pad = _sc_gather()(x, src)
    y_pad = _grouped_mlp(nf, buf, nbu, nfetch, iv, islot, ibuf, prslot,
                         x_pad, wrow, Wg_t, Wu_t, Wd_t, Wg_i, Wu_i, Wd_i)
    shared = _shared_mlp(x, Wg_s, Wu_s, Wd_s)
    out = _sc_combine()(y_pad, shared, pos[:, 0], pos[:, 1])
    return out.reshape(orig)
